# Initial kernel scaffold; baseline (speedup 1.0000x reference)
import jax
import jax.numpy as jnp
from jax.experimental import pallas as pl


def kernel(X, pos_enc, edge_index, encW1, encb1, encW2, encb2, posW, posb, qW, qb, kW, kb, vW, vb, oW, ob, bn1g, bn1b, f1W, f1b, f2W, f2b, bn2g, bn2b, pW1, pb1, pW2, pb2, pW3, pb3):
    raise NotImplementedError("write your pallas kernel here")


# speedup vs baseline: 1.0809x; 1.0809x over previous
import jax
import jax.numpy as jnp
from jax import lax
from jax.experimental import pallas as pl
from jax.experimental.pallas import tpu as pltpu
from jax.experimental.pallas import tpu_sc as plsc

N = 10000
E = 320000
IN = 128
HID = 128
POS = 8
L = 4
H = 8
HD = 16

NTILES = 32
EPT = E // NTILES
WIN = 64
NW = -(-EPT // WIN)
EOUT = E + WIN

R = 400
NB = N // R


def _rot8(x, perm8):
    return x.at[perm8].get(mode="promise_in_bounds")


def _sddmm_body(q_hbm, k_hbm, src_hbm, dst_hbm, out_hbm,
                srcw, dstw, qrows, krows, abuf, sem):
    w = lax.axis_index("c") * 16 + lax.axis_index("s")
    e0 = w * EPT
    perm8 = (lax.iota(jnp.int32, 16) + 8) & 15

    def issue(j):
        b = j & 1
        woff = e0 + j * WIN
        pltpu.sync_copy(src_hbm.at[pl.ds(woff, WIN)], srcw.at[b])
        pltpu.sync_copy(dst_hbm.at[pl.ds(woff, WIN)], dstw.at[b])
        pltpu.make_async_copy(q_hbm.at[srcw.at[b]], qrows.at[b], sem).start()
        pltpu.make_async_copy(k_hbm.at[dstw.at[b]], krows.at[b], sem).start()

    issue(0)

    def win_body(j, carry):
        b = j & 1
        woff = e0 + j * WIN
        pltpu.make_async_copy(q_hbm.at[srcw.at[b]], qrows.at[b], sem).wait()
        pltpu.make_async_copy(k_hbm.at[dstw.at[b]], krows.at[b], sem).wait()

        @pl.when(j + 1 < NW)
        def _():
            issue(j + 1)

        def e_body(loc, c2):
            s = []
            for c in range(8):
                p = (qrows[b, loc, pl.ds(c * 16, 16)]
                     * krows[b, loc, pl.ds(c * 16, 16)])
                s.append(p + _rot8(p, perm8))
            t = [s[0] + s[1], s[2] + s[3], s[4] + s[5], s[6] + s[7]]
            u = [t[0] + t[1], t[2] + t[3]]
            a = u[0] + u[1]
            abuf[pl.ds(loc * 16, 16)] = a
            return c2

        lax.fori_loop(0, WIN, e_body, jnp.int32(0))
        pltpu.sync_copy(abuf, out_hbm.at[pl.ds(woff * 16, WIN * 16)])
        return carry

    lax.fori_loop(0, NW, win_body, jnp.int32(0))


def _sddmm(q, k, src_pad, dst_pad):
    mesh = plsc.VectorSubcoreMesh(core_axis_name="c", subcore_axis_name="s")
    f = pl.kernel(
        _sddmm_body,
        out_type=jax.ShapeDtypeStruct((EOUT * 16,), jnp.float32),
        mesh=mesh,
        scratch_types=[
            pltpu.VMEM((2, WIN), jnp.int32),
            pltpu.VMEM((2, WIN), jnp.int32),
            pltpu.VMEM((2, WIN, HID), jnp.float32),
            pltpu.VMEM((2, WIN, HID), jnp.float32),
            pltpu.VMEM((WIN * 16,), jnp.float32),
            pltpu.SemaphoreType.DMA,
        ],
    )
    return f(q, k, src_pad, dst_pad).reshape(EOUT, 16)


def _expsub_body(a_hbm, m_hbm, src_hbm, out_hbm, srcw, abuf, mrows, ebuf, sem):
    w = lax.axis_index("c") * 16 + lax.axis_index("s")
    e0 = w * EPT

    def issue(j):
        b = j & 1
        woff = e0 + j * WIN
        pltpu.sync_copy(src_hbm.at[pl.ds(woff, WIN)], srcw.at[b])
        pltpu.make_async_copy(a_hbm.at[pl.ds(woff * 16, WIN * 16)], abuf.at[b], sem).start()
        pltpu.make_async_copy(m_hbm.at[srcw.at[b]], mrows.at[b], sem).start()

    issue(0)

    def win_body(j, carry):
        b = j & 1
        woff = e0 + j * WIN
        pltpu.make_async_copy(a_hbm.at[pl.ds(woff * 16, WIN * 16)], abuf.at[b], sem).wait()
        pltpu.make_async_copy(m_hbm.at[srcw.at[b]], mrows.at[b], sem).wait()

        @pl.when(j + 1 < NW)
        def _():
            issue(j + 1)

        def e_body(loc, c2):
            a = abuf[b, pl.ds(loc * 16, 16)]
            m = mrows[b, loc, pl.ds(0, 16)]
            ebuf[pl.ds(loc * 16, 16)] = jnp.exp(a - m)
            return c2

        lax.fori_loop(0, WIN, e_body, jnp.int32(0))
        pltpu.sync_copy(ebuf, out_hbm.at[pl.ds(woff * 16, WIN * 16)])
        return carry

    lax.fori_loop(0, NW, win_body, jnp.int32(0))


def _expsub(attn16, m16, src_pad):
    mesh = plsc.VectorSubcoreMesh(core_axis_name="c", subcore_axis_name="s")
    f = pl.kernel(
        _expsub_body,
        out_type=jax.ShapeDtypeStruct((EOUT * 16,), jnp.float32),
        mesh=mesh,
        scratch_types=[
            pltpu.VMEM((2, WIN), jnp.int32),
            pltpu.VMEM((2, WIN * 16), jnp.float32),
            pltpu.VMEM((2, WIN, HID), jnp.float32),
            pltpu.VMEM((WIN * 16,), jnp.float32),
            pltpu.SemaphoreType.DMA,
        ],
    )
    return f(attn16.reshape(-1), m16, src_pad).reshape(EOUT, 16)


def _msg_body(e_hbm, den_hbm, src_hbm, dst_hbm, v_hbm, out_hbm,
              srcw, dstw, ebuf, drows, vrows, mbuf, sem):
    w = lax.axis_index("c") * 16 + lax.axis_index("s")
    e0 = w * EPT

    def issue(j):
        b = j & 1
        woff = e0 + j * WIN
        pltpu.sync_copy(src_hbm.at[pl.ds(woff, WIN)], srcw.at[b])
        pltpu.sync_copy(dst_hbm.at[pl.ds(woff, WIN)], dstw.at[b])
        pltpu.make_async_copy(e_hbm.at[pl.ds(woff * 16, WIN * 16)], ebuf.at[b], sem).start()
        pltpu.make_async_copy(den_hbm.at[srcw.at[b]], drows.at[b], sem).start()
        pltpu.make_async_copy(v_hbm.at[dstw.at[b]], vrows.at[b], sem).start()

    issue(0)

    def win_body(j, carry):
        b = j & 1
        woff = e0 + j * WIN
        pltpu.make_async_copy(e_hbm.at[pl.ds(woff * 16, WIN * 16)], ebuf.at[b], sem).wait()
        pltpu.make_async_copy(den_hbm.at[srcw.at[b]], drows.at[b], sem).wait()
        pltpu.make_async_copy(v_hbm.at[dstw.at[b]], vrows.at[b], sem).wait()

        @pl.when(j + 1 < NW)
        def _():
            issue(j + 1)

        def e_body(loc, c2):
            ev = ebuf[b, pl.ds(loc * 16, 16)]
            dv = drows[b, loc, pl.ds(0, 16)]
            a = ev / dv
            for c in range(8):
                vc = vrows[b, loc, pl.ds(c * 16, 16)]
                mbuf[pl.ds(loc * 128 + c * 16, 16)] = a * vc
            return c2

        lax.fori_loop(0, WIN, e_body, jnp.int32(0))
        pltpu.sync_copy(mbuf, out_hbm.at[pl.ds(woff * 128, WIN * 128)])
        return carry

    lax.fori_loop(0, NW, win_body, jnp.int32(0))


def _msg(e16, den16, src_pad, dst_pad, v):
    mesh = plsc.VectorSubcoreMesh(core_axis_name="c", subcore_axis_name="s")
    f = pl.kernel(
        _msg_body,
        out_type=jax.ShapeDtypeStruct((EOUT * 128,), jnp.float32),
        mesh=mesh,
        scratch_types=[
            pltpu.VMEM((2, WIN), jnp.int32),
            pltpu.VMEM((2, WIN), jnp.int32),
            pltpu.VMEM((2, WIN * 16), jnp.float32),
            pltpu.VMEM((2, WIN, HID), jnp.float32),
            pltpu.VMEM((2, WIN, HID), jnp.float32),
            pltpu.VMEM((WIN * 128,), jnp.float32),
            pltpu.SemaphoreType.DMA,
        ],
    )
    return f(e16.reshape(-1), den16, src_pad, dst_pad, v).reshape(EOUT, 128)


def _mm(a, b):
    return jnp.dot(a, b, preferred_element_type=jnp.float32)


def _full(shape):
    nd = len(shape)
    return pl.BlockSpec(shape, lambda *_: (0,) * nd)


def _rows(width):
    return pl.BlockSpec((R, width), lambda *args: (args[-1], 0))


def _encqkv_body(x_ref, pos_ref, w1, b1, w2, b2, pw, pb,
                 qw, qb, kw, kb, vw, vb,
                 h_ref, q_ref, k_ref, v_ref):
    scaling = HD ** (-0.5)
    x = x_ref[...]
    t = _mm(x, w1[...]) + b1[...]
    t = jnp.where(t > 0, t, 0.1 * t)
    h = _mm(t, w2[...]) + b2[...]
    h = h + _mm(pos_ref[...], pw[...]) + pb[...]
    h_ref[...] = h
    q_ref[...] = (_mm(h, qw[...]) + qb[...]) * scaling
    k_ref[...] = _mm(h, kw[...]) + kb[...]
    v_ref[...] = _mm(h, vw[...]) + vb[...]


def _encqkv(X, pos, w1, b1, w2, b2, pw, pb, qw, qb, kw, kb, vw, vb):
    return pl.pallas_call(
        _encqkv_body,
        grid=(NB,),
        in_specs=[
            _rows(IN), _rows(POS),
            _full((IN, HID)), _full((1, HID)), _full((HID, HID)), _full((1, HID)),
            _full((POS, HID)), _full((1, HID)),
            _full((HID, HID)), _full((1, HID)),
            _full((HID, HID)), _full((1, HID)),
            _full((HID, HID)), _full((1, HID)),
        ],
        out_specs=[_rows(HID)] * 4,
        out_shape=[jax.ShapeDtypeStruct((N, HID), jnp.float32)] * 4,
    )(X, pos, w1, b1, w2, b2, pw, pb, qw, qb, kw, kb, vw, vb)


def _qkv_body(h_ref, qw, qb, kw, kb, vw, vb, q_ref, k_ref, v_ref):
    scaling = HD ** (-0.5)
    h = h_ref[...]
    q_ref[...] = (_mm(h, qw[...]) + qb[...]) * scaling
    k_ref[...] = _mm(h, kw[...]) + kb[...]
    v_ref[...] = _mm(h, vw[...]) + vb[...]


def _qkv(h, qw, qb, kw, kb, vw, vb):
    return pl.pallas_call(
        _qkv_body,
        grid=(NB,),
        in_specs=[_rows(HID)] + [_full((HID, HID)), _full((1, HID))] * 3,
        out_specs=[_rows(HID)] * 3,
        out_shape=[jax.ShapeDtypeStruct((N, HID), jnp.float32)] * 3,
    )(h, qw, qb, kw, kb, vw, vb)


def _oproj_body(aout_ref, h_ref, ow, ob, t_ref):
    t_ref[...] = _mm(aout_ref[...], ow[...]) + ob[...] + h_ref[...]


def _oproj(aout, h, ow, ob):
    return pl.pallas_call(
        _oproj_body,
        grid=(NB,),
        in_specs=[_rows(HID), _rows(HID), _full((HID, HID)), _full((1, HID))],
        out_specs=_rows(HID),
        out_shape=jax.ShapeDtypeStruct((N, HID), jnp.float32),
    )(aout, h, ow, ob)


def _ffn_body(h_ref, f1w, f1b, f2w, f2b, u_ref):
    h = h_ref[...]
    hf = _mm(jnp.maximum(_mm(h, f1w[...]) + f1b[...], 0.0), f2w[...]) + f2b[...]
    u_ref[...] = h + hf


def _ffn(h, f1w, f1b, f2w, f2b):
    return pl.pallas_call(
        _ffn_body,
        grid=(NB,),
        in_specs=[_rows(HID), _full((HID, 2 * HID)), _full((1, 2 * HID)),
                  _full((2 * HID, HID)), _full((1, HID))],
        out_specs=_rows(HID),
        out_shape=jax.ShapeDtypeStruct((N, HID), jnp.float32),
    )(h, f1w, f1b, f2w, f2b)


def _head_body(p_ref, w1, b1, w2, b2, w3, b3, o_ref):
    t = jnp.maximum(_mm(p_ref[...], w1[...]) + b1[...], 0.0)
    t = jnp.maximum(_mm(t, w2[...]) + b2[...], 0.0)
    o_ref[...] = _mm(t, w3[...]) + b3[...]


def _head(pooled, w1, b1, w2, b2, w3, b3):
    return pl.pallas_call(
        _head_body,
        in_specs=[_full((1, HID)),
                  _full((HID, HID // 2)), _full((1, HID // 2)),
                  _full((HID // 2, HID // 4)), _full((1, HID // 4)),
                  _full((HID // 4, 1)), _full((1, 1))],
        out_specs=_full((1, 1)),
        out_shape=jax.ShapeDtypeStruct((1, 1), jnp.float32),
    )(pooled, w1, b1, w2, b2, w3, b3)


def kernel(X, pos_enc, edge_index, encW1, encb1, encW2, encb2, posW, posb,
           qW, qb, kW, kb, vW, vb, oW, ob, bn1g, bn1b, f1W, f1b, f2W, f2b,
           bn2g, bn2b, pW1, pb1, pW2, pb2, pW3, pb3):
    src = edge_index[0]
    dst = edge_index[1]
    src_pad = jnp.concatenate([src, jnp.zeros((2 * WIN,), jnp.int32)])
    dst_pad = jnp.concatenate([dst, jnp.zeros((2 * WIN,), jnp.int32)])
    b = lambda x: x.reshape(1, -1)

    def bn(x, g, bb):
        m = x.mean(axis=0)
        v = x.var(axis=0)
        return (x - m) / jnp.sqrt(v + 1e-5) * g + bb

    h, q, k, v = _encqkv(
        X, pos_enc, encW1, b(encb1), encW2, b(encb2), posW, b(posb),
        qW[0], b(qb[0]), kW[0], b(kb[0]), vW[0], b(vb[0]))

    for l in range(L):
        attn16 = _sddmm(q, k, src_pad, dst_pad)
        attn = attn16[:E, :8]
        m = jax.ops.segment_max(attn, src, num_segments=N)
        m16 = jnp.pad(jnp.concatenate([m, m], axis=1), ((0, 0), (0, HID - 16)))
        e16 = _expsub(attn16, m16, src_pad)
        den = jax.ops.segment_sum(e16[:E, :8], src, num_segments=N)
        den16 = jnp.pad(jnp.concatenate([den, den], axis=1), ((0, 0), (0, HID - 16)))
        msg = _msg(e16, den16, src_pad, dst_pad, v)
        aout = jax.ops.segment_sum(
            msg[:E].reshape(E, HD, H), src, num_segments=N).reshape(N, HID)
        t = _oproj(aout, h, oW[l], b(ob[l]))
        h1 = bn(t, bn1g[l], bn1b[l])
        u = _ffn(h1, f1W[l], b(f1b[l]), f2W[l], b(f2b[l]))
        h = bn(u, bn2g[l], bn2b[l])
        if l < L - 1:
            q, k, v = _qkv(h, qW[l + 1], b(qb[l + 1]), kW[l + 1], b(kb[l + 1]),
                           vW[l + 1], b(vb[l + 1]))

    pooled = jnp.sum(h, axis=0, keepdims=True)
    p = jnp.maximum(pooled @ pW1 + pb1, 0.0)
    p = jnp.maximum(p @ pW2 + pb2, 0.0)
    return p @ pW3 + pb3
